# baseline (speedup 1.0000x reference)
import functools

import jax
import jax.numpy as jnp
from jax import lax
from jax.experimental import pallas as pl

H = 128
W = 128
N = 2048

ROWS_PER_STEP = 8
P = ROWS_PER_STEP * W
K = 128
ALPHA_MIN = 1.0 / 255.0


def _raster_kernel(mx, my, ssx, ssy, srot, sop, ct, ucovxx, ucovxy, ucovyy,
                   bg, color_out, radii_out, pc_out):
    t = pl.program_id(0)

    @pl.when(t == 0)
    def _init():
        det = jnp.maximum(ucovxx[...] * ucovyy[...] - ucovxy[...] * ucovxy[...], 1e-8)
        mid = 0.5 * (ucovxx[...] + ucovyy[...])
        lam1 = mid + jnp.sqrt(jnp.maximum(mid * mid - det, 0.1))
        radii_out[...] = jnp.ceil(3.0 * jnp.sqrt(lam1)).astype(jnp.int32)
        pc_out[...] = jnp.zeros_like(pc_out)

    a = jnp.cos(srot[...])
    b = jnp.sin(srot[...])
    sx2 = ssx[...] * ssx[...]
    sy2 = ssy[...] * ssy[...]
    cov_xx = a * a * sx2 + b * b * sy2
    cov_xy = a * b * (sx2 - sy2)
    cov_yy = b * b * sx2 + a * a * sy2
    det = jnp.maximum(cov_xx * cov_yy - cov_xy * cov_xy, 1e-8)
    conic_a = cov_yy / det
    conic_b = -cov_xy / det
    conic_c = cov_xx / det

    lane = lax.broadcasted_iota(jnp.int32, (1, P), 1)
    px = (lane % W).astype(jnp.float32) + 0.5
    py = (t * ROWS_PER_STEP + lane // W).astype(jnp.float32) + 0.5

    ri = lax.broadcasted_iota(jnp.int32, (K, K), 0)
    ci = lax.broadcasted_iota(jnp.int32, (K, K), 1)
    mlow = (ri >= ci).astype(jnp.float32)

    logT = jnp.zeros((1, P), jnp.float32)
    img = jnp.zeros((3, P), jnp.float32)

    for c in range(N // K):
        sl = slice(c * K, (c + 1) * K)
        dx = px - mx[sl, :]
        dy = py - my[sl, :]
        power = (-0.5 * (conic_a[sl, :] * dx * dx + conic_c[sl, :] * dy * dy)
                 - conic_b[sl, :] * dx * dy)
        power = jnp.minimum(power, 0.0)
        araw = jnp.minimum(0.99, sop[sl, :] * jnp.exp(power))
        alpha = jnp.where(araw < ALPHA_MIN, 0.0, araw)

        cnt = jnp.sum((araw > ALPHA_MIN).astype(jnp.float32), axis=1,
                      keepdims=True)
        pc_out[sl, :] += cnt.astype(jnp.int32)

        logm = jnp.log1p(-alpha)
        s_incl = jnp.dot(mlow, logm, preferred_element_type=jnp.float32)
        t_prev = jnp.exp(logT + (s_incl - logm))
        w = t_prev * alpha
        img = img + jnp.dot(ct[:, sl], w, preferred_element_type=jnp.float32)
        logT = logT + jnp.sum(logm, axis=0, keepdims=True)

    img = img + bg[...] * jnp.exp(logT)
    for r in range(ROWS_PER_STEP):
        color_out[:, r, :] = img[:, r * W:(r + 1) * W]


@jax.jit
def kernel(means2D, colors, opacities, scales, rotations, depths, background):
    order = jnp.argsort(depths)
    col1 = lambda x: x.reshape(N, 1).astype(jnp.float32)

    s_means = means2D[order]
    s_scales = scales[order]
    s_rot = rotations[order]
    s_op = opacities[order, 0]
    ct = colors[order].T

    ua = jnp.cos(rotations)
    ub = jnp.sin(rotations)
    usx2 = scales[:, 0] ** 2
    usy2 = scales[:, 1] ** 2
    ucovxx = ua * ua * usx2 + ub * ub * usy2
    ucovxy = ua * ub * (usx2 - usy2)
    ucovyy = ub * ub * usx2 + ua * ua * usy2

    grid = (H // ROWS_PER_STEP,)
    full = lambda shape: pl.BlockSpec(shape, lambda t: (0,) * len(shape))
    out = pl.pallas_call(
        _raster_kernel,
        grid=grid,
        in_specs=[full((N, 1))] * 6 + [full((3, N))] + [full((N, 1))] * 3
                 + [full((3, 1))],
        out_specs=[
            pl.BlockSpec((3, ROWS_PER_STEP, W), lambda t: (0, t, 0)),
            full((N, 1)),
            full((N, 1)),
        ],
        out_shape=[
            jax.ShapeDtypeStruct((3, H, W), jnp.float32),
            jax.ShapeDtypeStruct((N, 1), jnp.int32),
            jax.ShapeDtypeStruct((N, 1), jnp.int32),
        ],
    )(col1(s_means[:, 0]), col1(s_means[:, 1]),
      col1(s_scales[:, 0]), col1(s_scales[:, 1]),
      col1(s_rot), col1(s_op), ct,
      col1(ucovxx), col1(ucovxy), col1(ucovyy),
      background.reshape(3, 1))

    color, radii, pc_sorted = out
    radii = radii.reshape(N)
    pix_covered = jnp.zeros((N,), jnp.int32).at[order].set(pc_sorted.reshape(N))
    return color, radii, pix_covered


# speedup vs baseline: 1.3040x; 1.3040x over previous
import jax
import jax.numpy as jnp
from jax import lax
from jax.experimental import pallas as pl
from jax.experimental.pallas import tpu as pltpu

H = 128
W = 128
N = 2048

ROWS_PER_STEP = 8
P = ROWS_PER_STEP * W
K = 128
ALPHA_MIN = 1.0 / 255.0


def _raster_kernel(mx, my, ssx, ssy, srot, sop, ct, ucovxx, ucovxy, ucovyy,
                   bg, color_out, radii_out, pc_out, ca, cb, cc, lop):
    t = pl.program_id(0)

    @pl.when(t == 0)
    def _init():
        det = jnp.maximum(ucovxx[...] * ucovyy[...] - ucovxy[...] * ucovxy[...], 1e-8)
        mid = 0.5 * (ucovxx[...] + ucovyy[...])
        lam1 = mid + jnp.sqrt(jnp.maximum(mid * mid - det, 0.1))
        radii_out[...] = jnp.ceil(3.0 * jnp.sqrt(lam1)).astype(jnp.int32)
        pc_out[...] = jnp.zeros_like(pc_out)

        a = jnp.cos(srot[...])
        b = jnp.sin(srot[...])
        sx2 = ssx[...] * ssx[...]
        sy2 = ssy[...] * ssy[...]
        cov_xx = a * a * sx2 + b * b * sy2
        cov_xy = a * b * (sx2 - sy2)
        cov_yy = b * b * sx2 + a * a * sy2
        sdet = jnp.maximum(cov_xx * cov_yy - cov_xy * cov_xy, 1e-8)
        ca[...] = -0.5 * cov_yy / sdet
        cb[...] = cov_xy / sdet
        cc[...] = -0.5 * cov_xx / sdet
        lop[...] = jnp.log(sop[...])

    lane = lax.broadcasted_iota(jnp.int32, (1, P), 1)
    px = (lane % W).astype(jnp.float32) + 0.5
    py = (t * ROWS_PER_STEP + lane // W).astype(jnp.float32) + 0.5

    ri = lax.broadcasted_iota(jnp.int32, (K, K), 0)
    ci = lax.broadcasted_iota(jnp.int32, (K, K), 1)
    mstrict = (ri > ci).astype(jnp.float32)

    logT = jnp.zeros((1, P), jnp.float32)
    img = jnp.zeros((3, P), jnp.float32)

    for c in range(N // K):
        sl = slice(c * K, (c + 1) * K)
        dx = px - mx[sl, :]
        dy = py - my[sl, :]
        power = ca[sl, :] * (dx * dx) + cc[sl, :] * (dy * dy) + cb[sl, :] * (dx * dy)
        power = jnp.minimum(power, 0.0)
        araw = jnp.minimum(0.99, jnp.exp(power + lop[sl, :]))
        m = araw > ALPHA_MIN
        alpha = jnp.where(m, araw, 0.0)

        cnt = jnp.sum(m.astype(jnp.float32), axis=1, keepdims=True)
        pc_out[sl, :] += cnt.astype(jnp.int32)

        logm = jnp.log1p(-alpha)
        s_excl = jnp.dot(mstrict, logm, preferred_element_type=jnp.float32)
        w = jnp.exp(logT + s_excl) * alpha
        img = img + jnp.dot(ct[:, sl], w, preferred_element_type=jnp.float32)
        logT = logT + jnp.sum(logm, axis=0, keepdims=True)

    img = img + bg[...] * jnp.exp(logT)
    for r in range(ROWS_PER_STEP):
        color_out[:, r, :] = img[:, r * W:(r + 1) * W]


@jax.jit
def kernel(means2D, colors, opacities, scales, rotations, depths, background):
    order = jnp.argsort(depths)
    col1 = lambda x: x.reshape(N, 1).astype(jnp.float32)

    s_means = means2D[order]
    s_scales = scales[order]
    s_rot = rotations[order]
    s_op = opacities[order, 0]
    ct = colors[order].T

    ua = jnp.cos(rotations)
    ub = jnp.sin(rotations)
    usx2 = scales[:, 0] ** 2
    usy2 = scales[:, 1] ** 2
    ucovxx = ua * ua * usx2 + ub * ub * usy2
    ucovxy = ua * ub * (usx2 - usy2)
    ucovyy = ub * ub * usx2 + ua * ua * usy2

    grid = (H // ROWS_PER_STEP,)
    full = lambda shape: pl.BlockSpec(shape, lambda t: (0,) * len(shape))
    out = pl.pallas_call(
        _raster_kernel,
        grid=grid,
        in_specs=[full((N, 1))] * 6 + [full((3, N))] + [full((N, 1))] * 3
                 + [full((3, 1))],
        out_specs=[
            pl.BlockSpec((3, ROWS_PER_STEP, W), lambda t: (0, t, 0)),
            full((N, 1)),
            full((N, 1)),
        ],
        out_shape=[
            jax.ShapeDtypeStruct((3, H, W), jnp.float32),
            jax.ShapeDtypeStruct((N, 1), jnp.int32),
            jax.ShapeDtypeStruct((N, 1), jnp.int32),
        ],
        scratch_shapes=[pltpu.VMEM((N, 1), jnp.float32)] * 4,
    )(col1(s_means[:, 0]), col1(s_means[:, 1]),
      col1(s_scales[:, 0]), col1(s_scales[:, 1]),
      col1(s_rot), col1(s_op), ct,
      col1(ucovxx), col1(ucovxy), col1(ucovyy),
      background.reshape(3, 1))

    color, radii, pc_sorted = out
    radii = radii.reshape(N)
    pix_covered = jnp.zeros((N,), jnp.int32).at[order].set(pc_sorted.reshape(N))
    return color, radii, pix_covered
